# baseline (speedup 1.0000x reference)
import functools

import jax
import jax.numpy as jnp
from jax import lax
from jax.experimental import pallas as pl
from jax.experimental.pallas import tpu as pltpu
from jax.experimental.pallas import tpu_sc as plsc

_B = 16384
_D = 64
_NIDX = 7
_NC = 2
_NS = 16
_L = 16
_NW = _NC * _NS
_BPW = _B // _NW
_CH = 128
_NCHUNK = _BPW // _CH


def _log_sigmoid(x):
    t = jnp.exp(-jnp.abs(x))
    s = t / (2.0 + t)
    s2 = s * s
    poly = 1.0 + s2 * (1.0 / 3.0 + s2 * (1.0 / 5.0 + s2 * (1.0 / 7.0 + s2 * (1.0 / 9.0))))
    return jnp.minimum(x, 0.0) - 2.0 * s * poly


def _sc_body(batchT_hbm, w1_hbm, w2_hbm, out_hbm, idx_v, r0_v, r1_v, r2_v,
             r3_v, r4_v, r5_v, r6_v, acc_v, sem):
    rows = (r0_v, r1_v, r2_v, r3_v, r4_v, r5_v, r6_v)
    cid = lax.axis_index("c")
    sid = lax.axis_index("s")
    wid = sid * _NC + cid
    base = pl.multiple_of(wid * _BPW, _CH)

    acc = jnp.zeros((_L,), jnp.float32)
    for c in range(_NCHUNK):
        cbase = pl.multiple_of(base + c * _CH, _CH)
        pltpu.sync_copy(batchT_hbm.at[:, pl.ds(cbase, _CH)], idx_v)
        copies = [pltpu.async_copy(w1_hbm.at[idx_v.at[0]], rows[0], sem)]
        for k in range(1, _NIDX):
            copies.append(
                pltpu.async_copy(w2_hbm.at[idx_v.at[k]], rows[k], sem))
        for cp in copies:
            cp.wait()

        for g in range(_CH // _L):
            r = lax.iota(jnp.int32, _L) + g * _L

            def dbody(d, accs, r=r):
                dcol = jnp.full((_L,), d, jnp.int32)
                vi = plsc.load_gather(rows[0], [r, dcol])
                vj = plsc.load_gather(rows[1], [r, dcol])
                out = [accs[0] + vi * vj]
                for k in range(5):
                    nk = plsc.load_gather(rows[2 + k], [r, dcol])
                    out.append(accs[k + 1] + vi * nk)
                return tuple(out)

            zeros6 = tuple(jnp.zeros((_L,), jnp.float32) for _ in range(6))
            dots = lax.fori_loop(0, _D, dbody, zeros6)
            acc = acc + _log_sigmoid(dots[0])
            for k in range(5):
                acc = acc + _log_sigmoid(-dots[k + 1])

    acc_v[...] = acc
    pltpu.sync_copy(acc_v, out_hbm.at[wid])


@functools.partial(jax.jit, static_argnames=())
def _sc_loss_partials(batchT, W1, W2):
    mesh = plsc.VectorSubcoreMesh(core_axis_name="c", subcore_axis_name="s")
    f = pl.kernel(
        _sc_body,
        out_type=jax.ShapeDtypeStruct((_NW, _L), jnp.float32),
        mesh=mesh,
        scratch_types=[
            pltpu.VMEM((_NIDX, _CH), jnp.int32),
        ] + [pltpu.VMEM((_CH, _D), jnp.float32) for _ in range(_NIDX)] + [
            pltpu.VMEM((_L,), jnp.float32),
            pltpu.SemaphoreType.DMA,
        ],
        compiler_params=pltpu.CompilerParams(
            needs_layout_passes=False, use_tc_tiling_on_sc=False),
    )
    return f(batchT, W1, W2)


def kernel(batch, W1, W2):
    batchT = batch.astype(jnp.int32).T
    partials = _sc_loss_partials(batchT, W1, W2)
    return -jnp.sum(partials) / jnp.float32(_B)
